# baseline (speedup 1.0000x reference)
import functools

import jax
import jax.numpy as jnp
import numpy as np
from jax import lax
from jax.experimental import pallas as pl
from jax.experimental.pallas import tpu as pltpu
from jax.experimental.pallas import tpu_sc as plsc

N = 100000
E = 3200000
IN_DIM = 3
H1 = 32
H2 = 16
LAT = 2

NC = 2
NS = 16
LANES = 128
G_STREAMS = 4
GROUP = LANES * G_STREAMS

ROWS_PER_SUB = N // NS
DEG_N = 100096
DEG_RPS = DEG_N // NS

L2_FULL = E // (NC * NS * GROUP)
L2_PIPE = 192
L2_BASE = L2_FULL * GROUP
L2_EXTRA = E - NC * NS * L2_BASE
L1_FULL = E // (NS * GROUP)
L1_PIPE = 388
L1_BASE = L1_FULL * GROUP
L1_EXTRA = E - NS * L1_BASE

DGROUP = 1024
DG_FULL = E // (NC * NS * DGROUP)
DG_PIPE = 96
DG_BASE = DG_FULL * DGROUP
DG_EXTRA_BASE = NC * NS * DG_BASE

TC_BLK = 4096
TC_GRID = (N + TC_BLK - 1) // TC_BLK
P_BLK = TC_BLK * H2 // 128
P_N = N * H2 // 128

_mesh = plsc.VectorSubcoreMesh(core_axis_name="c", subcore_axis_name="s")
_sc_params = pltpu.CompilerParams(use_tc_tiling_on_sc=False)


def _deg_body(ei, out, idx_a, idx_b, ones, zb, acc, isem_a, isem_b,
              ssem_a, ssem_b):
    c = lax.axis_index("c")
    s = lax.axis_index("s")
    w = c * NS + s
    o = jnp.ones((16,), jnp.float32)
    z = jnp.zeros((16,), jnp.float32)

    @pl.loop(0, LANES // 16)
    def _(i):
        ones[pl.ds(i * 16, 16)] = o

    @pl.loop(0, DEG_RPS // 16)
    def _(i):
        zb[pl.ds(i * 16, 16)] = z

    pltpu.sync_copy(zb, acc.at[pl.ds(s * DEG_RPS, DEG_RPS)])
    plsc.subcore_barrier()

    base = w * DG_BASE

    def iload(g, buf, sem):
        pltpu.async_copy(ei.at[1, pl.ds(base + g * DGROUP, DGROUP)], buf, sem)

    def drain_i(buf, sem):
        pltpu.make_async_copy(ei.at[1, pl.ds(0, DGROUP)], buf, sem).wait()

    def fire_s(buf, sem):
        for j in range(DGROUP // LANES):
            pltpu.async_copy(ones, acc.at[buf.at[pl.ds(j * LANES, LANES)]],
                             sem, add=True)

    def drain_s(sem):
        pltpu.make_async_copy(ei.at[1, pl.ds(0, DGROUP)], idx_a, sem).wait()

    def seq_group(off, buf):
        pltpu.sync_copy(ei.at[1, pl.ds(off, DGROUP)], buf)
        fire_s(buf, ssem_a)
        drain_s(ssem_a)

    iload(0, idx_a, isem_a)
    iload(1, idx_b, isem_b)

    @pl.loop(0, DG_PIPE // 2 - 1)
    def _(i):
        g0 = 2 * i
        drain_i(idx_a, isem_a)
        fire_s(idx_a, ssem_a)
        drain_i(idx_b, isem_b)
        fire_s(idx_b, ssem_b)
        drain_s(ssem_a)
        iload(g0 + 2, idx_a, isem_a)
        drain_s(ssem_b)
        iload(g0 + 3, idx_b, isem_b)

    drain_i(idx_a, isem_a)
    fire_s(idx_a, ssem_a)
    drain_i(idx_b, isem_b)
    fire_s(idx_b, ssem_b)
    drain_s(ssem_a)
    drain_s(ssem_b)

    @pl.loop(DG_PIPE, DG_FULL)
    def _(g):
        seq_group(base + g * DGROUP, idx_a)

    @pl.when(w < 21)
    def _():
        seq_group(DG_EXTRA_BASE + w * DGROUP, idx_a)

    plsc.subcore_barrier()
    pltpu.sync_copy(acc.at[pl.ds(s * DEG_RPS, DEG_RPS)],
                    out.at[c, pl.ds(s * DEG_RPS, DEG_RPS)])


_deg_call = pl.kernel(
    _deg_body,
    out_type=jax.ShapeDtypeStruct((2, DEG_N), jnp.float32),
    mesh=_mesh,
    scratch_types=[
        pltpu.VMEM((DGROUP,), jnp.int32),
        pltpu.VMEM((DGROUP,), jnp.int32),
        pltpu.VMEM((LANES,), jnp.float32),
        pltpu.VMEM((DEG_RPS,), jnp.float32),
        pltpu.VMEM_SHARED((DEG_N,), jnp.float32),
        pltpu.SemaphoreType.DMA,
        pltpu.SemaphoreType.DMA,
        pltpu.SemaphoreType.DMA,
        pltpu.SemaphoreType.DMA,
    ],
    compiler_params=_sc_params,
)


def _edge_body(table_a, table_b, ei, out_a, out_b,
               sa0, da0, sa1, da1, sb0, db0, sb1, db1, rows_a, rows_b, acc,
               gsem_a, gsem_b, ssem_a, ssem_b, isem_a, isem_b,
               *, split_features):
    c = lax.axis_index("c")
    s = lax.axis_index("s")
    if split_features:
        base = s * L1_BASE
        npipe, nfull, extra_base = L1_PIPE, L1_FULL, NS * L1_BASE
    else:
        base = (c * NS + s) * L2_BASE
        npipe, nfull, extra_base = L2_PIPE, L2_FULL, NC * NS * L2_BASE

    sl_init = pl.ds(s * ROWS_PER_SUB, ROWS_PER_SUB)
    if split_features:
        @pl.when(c == 0)
        def _():
            pltpu.sync_copy(table_a.at[sl_init], acc.at[sl_init])

        @pl.when(c == 1)
        def _():
            pltpu.sync_copy(table_b.at[sl_init], acc.at[sl_init])
    else:
        @pl.when(c == 0)
        def _():
            pltpu.sync_copy(table_a.at[sl_init], acc.at[sl_init])

        @pl.when(c == 1)
        def _():
            @pl.loop(0, GROUP)
            def _(i):
                rows_a[i, :] = jnp.zeros((16,), jnp.float32)

            @pl.loop(0, ROWS_PER_SUB // GROUP)
            def _(i):
                pltpu.sync_copy(
                    rows_a,
                    acc.at[pl.ds(s * ROWS_PER_SUB + i * GROUP, GROUP)])

            rem = ROWS_PER_SUB % GROUP
            if rem:
                pltpu.sync_copy(
                    rows_a.at[pl.ds(0, rem)],
                    acc.at[pl.ds(s * ROWS_PER_SUB + ROWS_PER_SUB - rem, rem)],
                )

    plsc.subcore_barrier()

    def iload(g, sbuf, dbuf, sem):
        pltpu.async_copy(ei.at[0, pl.ds(base + g * GROUP, GROUP)], sbuf, sem)
        pltpu.async_copy(ei.at[1, pl.ds(base + g * GROUP, GROUP)], dbuf, sem)

    def drain_i(sbuf, dbuf, sem):
        pltpu.make_async_copy(ei.at[0, pl.ds(0, GROUP)], sbuf, sem).wait()
        pltpu.make_async_copy(ei.at[1, pl.ds(0, GROUP)], dbuf, sem).wait()

    def fire_g(sbuf, rows, sem):
        @pl.when(c == 0)
        def _():
            for j in range(G_STREAMS):
                pltpu.async_copy(table_a.at[sbuf.at[pl.ds(j * LANES, LANES)]],
                                 rows.at[pl.ds(j * LANES, LANES)], sem)

        @pl.when(c == 1)
        def _():
            for j in range(G_STREAMS):
                pltpu.async_copy(table_b.at[sbuf.at[pl.ds(j * LANES, LANES)]],
                                 rows.at[pl.ds(j * LANES, LANES)], sem)

    def fire_s(dbuf, rows, sem):
        for j in range(G_STREAMS):
            pltpu.async_copy(rows.at[pl.ds(j * LANES, LANES)],
                             acc.at[dbuf.at[pl.ds(j * LANES, LANES)]], sem,
                             add=True)

    def drain(rows, sem):
        pltpu.make_async_copy(table_a.at[pl.ds(0, GROUP)], rows, sem).wait()

    def seq_group(off, sbuf, dbuf, rows):
        pltpu.sync_copy(ei.at[0, pl.ds(off, GROUP)], sbuf)
        pltpu.sync_copy(ei.at[1, pl.ds(off, GROUP)], dbuf)
        fire_g(sbuf, rows, gsem_a)
        drain(rows, gsem_a)
        fire_s(dbuf, rows, ssem_a)
        drain(rows, ssem_a)

    iload(0, sa0, da0, isem_a)
    iload(1, sb0, db0, isem_b)
    drain_i(sa0, da0, isem_a)
    fire_g(sa0, rows_a, gsem_a)
    drain_i(sb0, db0, isem_b)
    fire_g(sb0, rows_b, gsem_b)
    iload(2, sa1, da1, isem_a)
    iload(3, sb1, db1, isem_b)

    @pl.loop(0, npipe // 4 - 1)
    def _(i):
        g = 4 * i
        drain(rows_a, gsem_a)
        fire_s(da0, rows_a, ssem_a)
        drain(rows_b, gsem_b)
        fire_s(db0, rows_b, ssem_b)
        drain(rows_a, ssem_a)
        drain_i(sa1, da1, isem_a)
        fire_g(sa1, rows_a, gsem_a)
        iload(g + 4, sa0, da0, isem_a)
        drain(rows_b, ssem_b)
        drain_i(sb1, db1, isem_b)
        fire_g(sb1, rows_b, gsem_b)
        iload(g + 5, sb0, db0, isem_b)
        drain(rows_a, gsem_a)
        fire_s(da1, rows_a, ssem_a)
        drain(rows_b, gsem_b)
        fire_s(db1, rows_b, ssem_b)
        drain(rows_a, ssem_a)
        drain_i(sa0, da0, isem_a)
        fire_g(sa0, rows_a, gsem_a)
        iload(g + 6, sa1, da1, isem_a)
        drain(rows_b, ssem_b)
        drain_i(sb0, db0, isem_b)
        fire_g(sb0, rows_b, gsem_b)
        iload(g + 7, sb1, db1, isem_b)

    drain(rows_a, gsem_a)
    fire_s(da0, rows_a, ssem_a)
    drain(rows_b, gsem_b)
    fire_s(db0, rows_b, ssem_b)
    drain(rows_a, ssem_a)
    drain_i(sa1, da1, isem_a)
    fire_g(sa1, rows_a, gsem_a)
    drain(rows_b, ssem_b)
    drain_i(sb1, db1, isem_b)
    fire_g(sb1, rows_b, gsem_b)
    drain(rows_a, gsem_a)
    fire_s(da1, rows_a, ssem_a)
    drain(rows_b, gsem_b)
    fire_s(db1, rows_b, ssem_b)
    drain(rows_a, ssem_a)
    drain(rows_b, ssem_b)

    @pl.loop(npipe, nfull)
    def _(g):
        seq_group(base + g * GROUP, sa0, da0, rows_a)

    if split_features:
        @pl.when(s < L1_EXTRA // GROUP)
        def _():
            seq_group(extra_base + s * GROUP, sa0, da0, rows_a)
    else:
        w = c * NS + s

        @pl.when(w < L2_EXTRA // GROUP)
        def _():
            seq_group(extra_base + w * GROUP, sa0, da0, rows_a)

    plsc.subcore_barrier()
    sl = pl.ds(s * ROWS_PER_SUB, ROWS_PER_SUB)

    @pl.when(c == 0)
    def _():
        pltpu.sync_copy(acc.at[sl], out_a.at[sl])

    @pl.when(c == 1)
    def _():
        pltpu.sync_copy(acc.at[sl], out_b.at[sl])


def _make_edge_call(split_features):
    return pl.kernel(
        functools.partial(_edge_body, split_features=split_features),
        out_type=(
            jax.ShapeDtypeStruct((N, H2), jnp.float32),
            jax.ShapeDtypeStruct((N, H2), jnp.float32),
        ),
        mesh=_mesh,
        scratch_types=(
            [pltpu.VMEM((GROUP,), jnp.int32) for _ in range(8)]
            + [
                pltpu.VMEM((GROUP, H2), jnp.float32),
                pltpu.VMEM((GROUP, H2), jnp.float32),
                pltpu.VMEM_SHARED((N, H2), jnp.float32),
            ]
            + [pltpu.SemaphoreType.DMA for _ in range(6)]
        ),
        compiler_params=_sc_params,
    )


_l1_call = _make_edge_call(split_features=True)
_l2_call = _make_edge_call(split_features=False)


def _mm(a, b):
    return jnp.dot(a, b, preferred_element_type=jnp.float32)


def _k2_body(deg_ref, x_ref, w1t_ref, g1a_ref, g1b_ref):
    dinv = lax.rsqrt(deg_ref[0:1, :] + deg_ref[1:2, :] + 1.0)
    xt = x_ref[...].T
    gt = _mm(w1t_ref[...], xt) * dinv
    g1a_ref[...] = gt[:H2].T
    g1b_ref[...] = gt[H2:].T


def _k4_body(deg_ref, s1a_ref, s1b_ref, b1c_ref, w2t_ref, g2_ref):
    dinv = lax.rsqrt(deg_ref[0:1, :] + deg_ref[1:2, :] + 1.0)
    ta = s1a_ref[...].T
    tb = s1b_ref[...].T
    b1c = b1c_ref[...]
    ha = jnp.maximum(ta * dinv + b1c[:H2], 0.0)
    hb = jnp.maximum(tb * dinv + b1c[H2:], 0.0)
    w2t = w2t_ref[...]
    g2t = (_mm(w2t[:, :H2], ha) + _mm(w2t[:, H2:], hb)) * dinv
    g2_ref[...] = g2t.T


def _k6_body(deg_ref, s2_ref, b2c_ref,
             wbt_ref, bbc_ref, wd1t_ref, bd1c_ref, wd2t_ref, bd2c_ref,
             wd3t_ref, bd3c_ref, out_ref):
    dinv = lax.rsqrt(deg_ref[0:1, :] + deg_ref[1:2, :] + 1.0)
    st = s2_ref[...].T
    h2 = jnp.maximum(st * dinv + b2c_ref[...], 0.0)
    z = _mm(wbt_ref[...], h2) + bbc_ref[...]
    d = jnp.maximum(_mm(wd1t_ref[...], z) + bd1c_ref[...], 0.0)
    d = jnp.maximum(_mm(wd2t_ref[...], d) + bd2c_ref[...], 0.0)
    ot = _mm(wd3t_ref[...], d) + bd3c_ref[...]
    out_ref[...] = ot.T


def _blk(*shape):
    nd = len(shape)
    if nd == 1:
        return pl.BlockSpec(shape, lambda i: (i,))
    return pl.BlockSpec(shape, lambda i: (i,) + (0,) * (nd - 1))


def _cols(*shape):
    return pl.BlockSpec(shape, lambda i: (0, i))


def _full(*shape):
    nd = len(shape)
    return pl.BlockSpec(shape, lambda i: (0,) * nd)


def kernel(x, edge_index, W1, b1, W2, b2, Wb, bb, Wd1, bd1, Wd2, bd2,
           Wd3, bd3):
    deg = _deg_call(edge_index)

    g1a, g1b = pl.pallas_call(
        _k2_body,
        grid=(TC_GRID,),
        in_specs=[
            _cols(2, TC_BLK),
            _blk(TC_BLK, IN_DIM),
            _full(H1, IN_DIM),
        ],
        out_specs=[_blk(TC_BLK, H2), _blk(TC_BLK, H2)],
        out_shape=[
            jax.ShapeDtypeStruct((N, H2), jnp.float32),
            jax.ShapeDtypeStruct((N, H2), jnp.float32),
        ],
    )(deg, x, W1.T)

    s1a, s1b = _l1_call(g1a, g1b, edge_index)

    g2 = pl.pallas_call(
        _k4_body,
        grid=(TC_GRID,),
        in_specs=[
            _cols(2, TC_BLK),
            _blk(TC_BLK, H2),
            _blk(TC_BLK, H2),
            _full(H1, 1),
            _full(H2, H1),
        ],
        out_specs=_blk(TC_BLK, H2),
        out_shape=jax.ShapeDtypeStruct((N, H2), jnp.float32),
    )(deg, s1a, s1b, b1.reshape(H1, 1), W2.T)

    s2a, s2b = _l2_call(g2, g2, edge_index)

    out = pl.pallas_call(
        _k6_body,
        grid=(TC_GRID,),
        in_specs=[
            _cols(2, TC_BLK),
            _blk(TC_BLK, H2),
            _full(H2, 1),
            _full(LAT, H2),
            _full(LAT, 1),
            _full(H2, LAT),
            _full(H2, 1),
            _full(H1, H2),
            _full(H1, 1),
            _full(IN_DIM, H1),
            _full(IN_DIM, 1),
        ],
        out_specs=_blk(TC_BLK, IN_DIM),
        out_shape=jax.ShapeDtypeStruct((N, IN_DIM), jnp.float32),
    )(deg, s2a + s2b, b2.reshape(H2, 1), Wb.T,
      bb.reshape(LAT, 1), Wd1.T, bd1.reshape(H2, 1), Wd2.T,
      bd2.reshape(H1, 1), Wd3.T, bd3.reshape(IN_DIM, 1))

    return out


# speedup vs baseline: 1.0348x; 1.0348x over previous
import functools

import jax
import jax.numpy as jnp
import numpy as np
from jax import lax
from jax.experimental import pallas as pl
from jax.experimental.pallas import tpu as pltpu
from jax.experimental.pallas import tpu_sc as plsc

N = 100000
E = 3200000
IN_DIM = 3
H1 = 32
H2 = 16
LAT = 2

NC = 2
NS = 16
LANES = 128
G_STREAMS = 4
GROUP = LANES * G_STREAMS

ROWS_PER_SUB = N // NS
DEG_N = 100096
DEG_RPS = DEG_N // NS

L2_FULL = E // (NC * NS * GROUP)
L2_PIPE = 192
L2_BASE = L2_FULL * GROUP
L2_EXTRA = E - NC * NS * L2_BASE
L1_FULL = E // (NS * GROUP)
L1_PIPE = 388
L1_BASE = L1_FULL * GROUP
L1_EXTRA = E - NS * L1_BASE

DGROUP = 1024
DG_FULL = E // (NC * NS * DGROUP)
DG_PIPE = 96
DG_BASE = DG_FULL * DGROUP
DG_EXTRA_BASE = NC * NS * DG_BASE

TC_BLK = 4096
TC_GRID = (N + TC_BLK - 1) // TC_BLK
P_BLK = TC_BLK * H2 // 128
P_N = N * H2 // 128

_mesh = plsc.VectorSubcoreMesh(core_axis_name="c", subcore_axis_name="s")
_sc_params = pltpu.CompilerParams(use_tc_tiling_on_sc=False)


def _deg_body(ei, out, idx_a, idx_b, ones, zb, acc, isem_a, isem_b,
              ssem_a, ssem_b):
    c = lax.axis_index("c")
    s = lax.axis_index("s")
    w = c * NS + s
    o = jnp.ones((16,), jnp.float32)
    z = jnp.zeros((16,), jnp.float32)

    @pl.loop(0, LANES // 16)
    def _(i):
        ones[pl.ds(i * 16, 16)] = o

    @pl.loop(0, DEG_RPS // 16)
    def _(i):
        zb[pl.ds(i * 16, 16)] = z

    pltpu.sync_copy(zb, acc.at[pl.ds(s * DEG_RPS, DEG_RPS)])
    plsc.subcore_barrier()

    base = w * DG_BASE

    def iload(g, buf, sem):
        pltpu.async_copy(ei.at[1, pl.ds(base + g * DGROUP, DGROUP)], buf, sem)

    def drain_i(buf, sem):
        pltpu.make_async_copy(ei.at[1, pl.ds(0, DGROUP)], buf, sem).wait()

    def fire_s(buf, sem):
        for j in range(DGROUP // LANES):
            pltpu.async_copy(ones, acc.at[buf.at[pl.ds(j * LANES, LANES)]],
                             sem, add=True)

    def drain_s(sem):
        pltpu.make_async_copy(ei.at[1, pl.ds(0, DGROUP)], idx_a, sem).wait()

    def seq_group(off, buf):
        pltpu.sync_copy(ei.at[1, pl.ds(off, DGROUP)], buf)
        fire_s(buf, ssem_a)
        drain_s(ssem_a)

    iload(0, idx_a, isem_a)
    iload(1, idx_b, isem_b)

    @pl.loop(0, DG_PIPE // 2 - 1)
    def _(i):
        g0 = 2 * i
        drain_i(idx_a, isem_a)
        fire_s(idx_a, ssem_a)
        drain_i(idx_b, isem_b)
        fire_s(idx_b, ssem_b)
        drain_s(ssem_a)
        iload(g0 + 2, idx_a, isem_a)
        drain_s(ssem_b)
        iload(g0 + 3, idx_b, isem_b)

    drain_i(idx_a, isem_a)
    fire_s(idx_a, ssem_a)
    drain_i(idx_b, isem_b)
    fire_s(idx_b, ssem_b)
    drain_s(ssem_a)
    drain_s(ssem_b)

    @pl.loop(DG_PIPE, DG_FULL)
    def _(g):
        seq_group(base + g * DGROUP, idx_a)

    @pl.when(w < 21)
    def _():
        seq_group(DG_EXTRA_BASE + w * DGROUP, idx_a)

    plsc.subcore_barrier()
    pltpu.sync_copy(acc.at[pl.ds(s * DEG_RPS, DEG_RPS)],
                    out.at[c, pl.ds(s * DEG_RPS, DEG_RPS)])


_deg_call = pl.kernel(
    _deg_body,
    out_type=jax.ShapeDtypeStruct((2, DEG_N), jnp.float32),
    mesh=_mesh,
    scratch_types=[
        pltpu.VMEM((DGROUP,), jnp.int32),
        pltpu.VMEM((DGROUP,), jnp.int32),
        pltpu.VMEM((LANES,), jnp.float32),
        pltpu.VMEM((DEG_RPS,), jnp.float32),
        pltpu.VMEM_SHARED((DEG_N,), jnp.float32),
        pltpu.SemaphoreType.DMA,
        pltpu.SemaphoreType.DMA,
        pltpu.SemaphoreType.DMA,
        pltpu.SemaphoreType.DMA,
    ],
    compiler_params=_sc_params,
)


def _edge_body(table_a, table_b, ei, out_a, out_b,
               sa0, da0, sa1, da1, sb0, db0, sb1, db1, rows_a, rows_b, acc,
               gsem_a, gsem_b, ssem_a, ssem_b, isem_a, isem_b,
               *, split_features):
    c = lax.axis_index("c")
    s = lax.axis_index("s")
    if split_features:
        base = s * L1_BASE
        npipe, nfull, extra_base = L1_PIPE, L1_FULL, NS * L1_BASE
    else:
        base = (c * NS + s) * L2_BASE
        npipe, nfull, extra_base = L2_PIPE, L2_FULL, NC * NS * L2_BASE

    sl_init = pl.ds(s * ROWS_PER_SUB, ROWS_PER_SUB)
    if split_features:
        @pl.when(c == 0)
        def _():
            pltpu.sync_copy(table_a.at[sl_init], acc.at[sl_init])

        @pl.when(c == 1)
        def _():
            pltpu.sync_copy(table_b.at[sl_init], acc.at[sl_init])
    else:
        @pl.when(c == 0)
        def _():
            pltpu.sync_copy(table_a.at[sl_init], acc.at[sl_init])

        @pl.when(c == 1)
        def _():
            @pl.loop(0, GROUP)
            def _(i):
                rows_a[i, :] = jnp.zeros((16,), jnp.float32)

            @pl.loop(0, ROWS_PER_SUB // GROUP)
            def _(i):
                pltpu.sync_copy(
                    rows_a,
                    acc.at[pl.ds(s * ROWS_PER_SUB + i * GROUP, GROUP)])

            rem = ROWS_PER_SUB % GROUP
            if rem:
                pltpu.sync_copy(
                    rows_a.at[pl.ds(0, rem)],
                    acc.at[pl.ds(s * ROWS_PER_SUB + ROWS_PER_SUB - rem, rem)],
                )

    plsc.subcore_barrier()

    def iload(g, sbuf, dbuf, sem):
        pltpu.async_copy(ei.at[0, pl.ds(base + g * GROUP, GROUP)], sbuf, sem)
        pltpu.async_copy(ei.at[1, pl.ds(base + g * GROUP, GROUP)], dbuf, sem)

    def drain_i(sbuf, dbuf, sem):
        pltpu.make_async_copy(ei.at[0, pl.ds(0, GROUP)], sbuf, sem).wait()
        pltpu.make_async_copy(ei.at[1, pl.ds(0, GROUP)], dbuf, sem).wait()

    def fire_g(sbuf, rows, sem):
        @pl.when(c == 0)
        def _():
            for j in range(G_STREAMS):
                pltpu.async_copy(table_a.at[sbuf.at[pl.ds(j * LANES, LANES)]],
                                 rows.at[pl.ds(j * LANES, LANES)], sem)

        @pl.when(c == 1)
        def _():
            for j in range(G_STREAMS):
                pltpu.async_copy(table_b.at[sbuf.at[pl.ds(j * LANES, LANES)]],
                                 rows.at[pl.ds(j * LANES, LANES)], sem)

    def fire_s(dbuf, rows, sem):
        for j in range(G_STREAMS):
            pltpu.async_copy(rows.at[pl.ds(j * LANES, LANES)],
                             acc.at[dbuf.at[pl.ds(j * LANES, LANES)]], sem,
                             add=True)

    def drain(rows, sem):
        pltpu.make_async_copy(table_a.at[pl.ds(0, GROUP)], rows, sem).wait()

    def seq_group(off, sbuf, dbuf, rows):
        pltpu.sync_copy(ei.at[0, pl.ds(off, GROUP)], sbuf)
        pltpu.sync_copy(ei.at[1, pl.ds(off, GROUP)], dbuf)
        fire_g(sbuf, rows, gsem_a)
        drain(rows, gsem_a)
        fire_s(dbuf, rows, ssem_a)
        drain(rows, ssem_a)

    iload(0, sa0, da0, isem_a)
    iload(1, sb0, db0, isem_b)
    drain_i(sa0, da0, isem_a)
    fire_g(sa0, rows_a, gsem_a)
    drain_i(sb0, db0, isem_b)
    fire_g(sb0, rows_b, gsem_b)
    iload(2, sa1, da1, isem_a)
    iload(3, sb1, db1, isem_b)

    @pl.loop(0, npipe // 4 - 1)
    def _(i):
        g = 4 * i
        drain(rows_a, gsem_a)
        fire_s(da0, rows_a, ssem_a)
        drain(rows_b, gsem_b)
        fire_s(db0, rows_b, ssem_b)
        drain(rows_a, ssem_a)
        drain_i(sa1, da1, isem_a)
        fire_g(sa1, rows_a, gsem_a)
        iload(g + 4, sa0, da0, isem_a)
        drain(rows_b, ssem_b)
        drain_i(sb1, db1, isem_b)
        fire_g(sb1, rows_b, gsem_b)
        iload(g + 5, sb0, db0, isem_b)
        drain(rows_a, gsem_a)
        fire_s(da1, rows_a, ssem_a)
        drain(rows_b, gsem_b)
        fire_s(db1, rows_b, ssem_b)
        drain(rows_a, ssem_a)
        drain_i(sa0, da0, isem_a)
        fire_g(sa0, rows_a, gsem_a)
        iload(g + 6, sa1, da1, isem_a)
        drain(rows_b, ssem_b)
        drain_i(sb0, db0, isem_b)
        fire_g(sb0, rows_b, gsem_b)
        iload(g + 7, sb1, db1, isem_b)

    drain(rows_a, gsem_a)
    fire_s(da0, rows_a, ssem_a)
    drain(rows_b, gsem_b)
    fire_s(db0, rows_b, ssem_b)
    drain(rows_a, ssem_a)
    drain_i(sa1, da1, isem_a)
    fire_g(sa1, rows_a, gsem_a)
    drain(rows_b, ssem_b)
    drain_i(sb1, db1, isem_b)
    fire_g(sb1, rows_b, gsem_b)
    drain(rows_a, gsem_a)
    fire_s(da1, rows_a, ssem_a)
    drain(rows_b, gsem_b)
    fire_s(db1, rows_b, ssem_b)
    drain(rows_a, ssem_a)
    drain(rows_b, ssem_b)

    @pl.loop(npipe, nfull)
    def _(g):
        seq_group(base + g * GROUP, sa0, da0, rows_a)

    if split_features:
        @pl.when(s < L1_EXTRA // GROUP)
        def _():
            seq_group(extra_base + s * GROUP, sa0, da0, rows_a)
    else:
        w = c * NS + s

        @pl.when(w < L2_EXTRA // GROUP)
        def _():
            seq_group(extra_base + w * GROUP, sa0, da0, rows_a)

    plsc.subcore_barrier()
    sl = pl.ds(s * ROWS_PER_SUB, ROWS_PER_SUB)

    @pl.when(c == 0)
    def _():
        pltpu.sync_copy(acc.at[sl], out_a.at[sl])

    @pl.when(c == 1)
    def _():
        pltpu.sync_copy(acc.at[sl], out_b.at[sl])


def _make_edge_call(split_features):
    return pl.kernel(
        functools.partial(_edge_body, split_features=split_features),
        out_type=(
            jax.ShapeDtypeStruct((N, H2), jnp.float32),
            jax.ShapeDtypeStruct((N, H2), jnp.float32),
        ),
        mesh=_mesh,
        scratch_types=(
            [pltpu.VMEM((GROUP,), jnp.int32) for _ in range(8)]
            + [
                pltpu.VMEM((GROUP, H2), jnp.float32),
                pltpu.VMEM((GROUP, H2), jnp.float32),
                pltpu.VMEM_SHARED((N, H2), jnp.float32),
            ]
            + [pltpu.SemaphoreType.DMA for _ in range(6)]
        ),
        compiler_params=_sc_params,
    )


_l1_call = _make_edge_call(split_features=True)
_l2_call = _make_edge_call(split_features=False)


def _mm(a, b):
    return jnp.dot(a, b, preferred_element_type=jnp.float32)


def _k2_body(deg_ref, x_ref, w1t_ref, g1a_ref, g1b_ref):
    dinv = lax.rsqrt(deg_ref[0:1, :] + deg_ref[1:2, :] + 1.0)
    xt = x_ref[...].T
    gt = _mm(w1t_ref[...], xt) * dinv
    g1a_ref[...] = gt[:H2].T
    g1b_ref[...] = gt[H2:].T


def _k4_body(deg_ref, s1a_ref, s1b_ref, b1c_ref, w2t_ref, g2_ref):
    dinv = lax.rsqrt(deg_ref[0:1, :] + deg_ref[1:2, :] + 1.0)
    ta = s1a_ref[...].T
    tb = s1b_ref[...].T
    b1c = b1c_ref[...]
    ha = jnp.maximum(ta * dinv + b1c[:H2], 0.0)
    hb = jnp.maximum(tb * dinv + b1c[H2:], 0.0)
    w2t = w2t_ref[...]
    g2t = (_mm(w2t[:, :H2], ha) + _mm(w2t[:, H2:], hb)) * dinv
    g2_ref[...] = g2t.T


def _k6_body(deg_ref, s2a_ref, s2b_ref, b2c_ref,
             wbt_ref, bbc_ref, wd1t_ref, bd1c_ref, wd2t_ref, bd2c_ref,
             wd3t_ref, bd3c_ref, out_ref):
    dinv = lax.rsqrt(deg_ref[0:1, :] + deg_ref[1:2, :] + 1.0)
    st = (s2a_ref[...] + s2b_ref[...]).T
    h2 = jnp.maximum(st * dinv + b2c_ref[...], 0.0)
    z = _mm(wbt_ref[...], h2) + bbc_ref[...]
    d = jnp.maximum(_mm(wd1t_ref[...], z) + bd1c_ref[...], 0.0)
    d = jnp.maximum(_mm(wd2t_ref[...], d) + bd2c_ref[...], 0.0)
    ot = _mm(wd3t_ref[...], d) + bd3c_ref[...]
    out_ref[...] = ot.T


def _blk(*shape):
    nd = len(shape)
    if nd == 1:
        return pl.BlockSpec(shape, lambda i: (i,))
    return pl.BlockSpec(shape, lambda i: (i,) + (0,) * (nd - 1))


def _cols(*shape):
    return pl.BlockSpec(shape, lambda i: (0, i))


def _full(*shape):
    nd = len(shape)
    return pl.BlockSpec(shape, lambda i: (0,) * nd)


def kernel(x, edge_index, W1, b1, W2, b2, Wb, bb, Wd1, bd1, Wd2, bd2,
           Wd3, bd3):
    deg = _deg_call(edge_index)

    g1a, g1b = pl.pallas_call(
        _k2_body,
        grid=(TC_GRID,),
        in_specs=[
            _cols(2, TC_BLK),
            _blk(TC_BLK, IN_DIM),
            _full(H1, IN_DIM),
        ],
        out_specs=[_blk(TC_BLK, H2), _blk(TC_BLK, H2)],
        out_shape=[
            jax.ShapeDtypeStruct((N, H2), jnp.float32),
            jax.ShapeDtypeStruct((N, H2), jnp.float32),
        ],
    )(deg, x, W1.T)

    s1a, s1b = _l1_call(g1a, g1b, edge_index)

    g2 = pl.pallas_call(
        _k4_body,
        grid=(TC_GRID,),
        in_specs=[
            _cols(2, TC_BLK),
            _blk(TC_BLK, H2),
            _blk(TC_BLK, H2),
            _full(H1, 1),
            _full(H2, H1),
        ],
        out_specs=_blk(TC_BLK, H2),
        out_shape=jax.ShapeDtypeStruct((N, H2), jnp.float32),
    )(deg, s1a, s1b, b1.reshape(H1, 1), W2.T)

    s2a, s2b = _l2_call(g2, g2, edge_index)

    out = pl.pallas_call(
        _k6_body,
        grid=(TC_GRID,),
        in_specs=[
            _cols(2, TC_BLK),
            _blk(TC_BLK, H2),
            _blk(TC_BLK, H2),
            _full(H2, 1),
            _full(LAT, H2),
            _full(LAT, 1),
            _full(H2, LAT),
            _full(H2, 1),
            _full(H1, H2),
            _full(H1, 1),
            _full(IN_DIM, H1),
            _full(IN_DIM, 1),
        ],
        out_specs=_blk(TC_BLK, IN_DIM),
        out_shape=jax.ShapeDtypeStruct((N, IN_DIM), jnp.float32),
    )(deg, s2a, s2b, b2.reshape(H2, 1), Wb.T,
      bb.reshape(LAT, 1), Wd1.T, bd1.reshape(H2, 1), Wd2.T,
      bd2.reshape(H1, 1), Wd3.T, bd3.reshape(IN_DIM, 1))

    return out
